# Initial kernel scaffold; baseline (speedup 1.0000x reference)
import jax
import jax.numpy as jnp
from jax.experimental import pallas as pl


def kernel(cumulative_positions, position_embeddings):
    raise NotImplementedError("write your pallas kernel here")


# speedup vs baseline: 2.7960x; 2.7960x over previous
import functools

import jax
import jax.numpy as jnp
from jax import lax
from jax.experimental import pallas as pl
from jax.experimental.pallas import tpu as pltpu
from jax.experimental.pallas import tpu_sc as plsc

D_MODEL = 64
NUM_WORKERS = 32
SUB = 128
K = 4
CHUNK = SUB * K


def _make_gather(n_rows):
    per_w = n_rows // NUM_WORKERS
    n_ch = per_w // CHUNK
    assert per_w % CHUNK == 0
    mesh = plsc.VectorSubcoreMesh(core_axis_name="c", subcore_axis_name="s")

    @functools.partial(
        pl.kernel,
        out_type=jax.ShapeDtypeStruct((n_rows, D_MODEL), jnp.float32),
        mesh=mesh,
        scratch_types=[
            pltpu.VMEM((K, SUB), jnp.int32),
            pltpu.VMEM((CHUNK, D_MODEL), jnp.float32),
            pltpu.SemaphoreType.DMA,
        ],
        compiler_params=pltpu.CompilerParams(use_tc_tiling_on_sc=False),
    )
    def k(table_hbm, idx_hbm, out_hbm, idx_v, rows_v, gsem):
        wid = lax.axis_index("s") * 2 + lax.axis_index("c")
        base = wid * (per_w // SUB)

        def body(g, carry):
            row0 = base + g * K
            pltpu.sync_copy(idx_hbm.at[pl.ds(row0, K)], idx_v)
            descs = [
                pltpu.async_copy(
                    table_hbm.at[idx_v.at[j]],
                    rows_v.at[pl.ds(j * SUB, SUB)],
                    gsem,
                )
                for j in range(K)
            ]
            for d in descs:
                d.wait()
            pltpu.sync_copy(rows_v, out_hbm.at[pl.ds(row0 * SUB, CHUNK)])
            return carry

        lax.fori_loop(0, n_ch, body, 0)

    return k


def kernel(cumulative_positions, position_embeddings):
    b, h = cumulative_positions.shape
    n = b * h
    idx2d = cumulative_positions.astype(jnp.int32).reshape(n // SUB, SUB)
    out = _make_gather(n)(position_embeddings, idx2d)
    return out.reshape(b, h, D_MODEL)


# speedup vs baseline: 2.8043x; 1.0030x over previous
import functools

import jax
import jax.numpy as jnp
from jax import lax
from jax.experimental import pallas as pl
from jax.experimental.pallas import tpu as pltpu
from jax.experimental.pallas import tpu_sc as plsc

D_MODEL = 64
NUM_WORKERS = 32
SUB = 128
K = 4
CHUNK = SUB * K
NBUF = 2


def _make_gather(n_rows):
    per_w = n_rows // NUM_WORKERS
    n_ch = per_w // CHUNK
    assert per_w % (CHUNK * NBUF) == 0
    mesh = plsc.VectorSubcoreMesh(core_axis_name="c", subcore_axis_name="s")

    @functools.partial(
        pl.kernel,
        out_type=jax.ShapeDtypeStruct((n_rows, D_MODEL), jnp.float32),
        mesh=mesh,
        scratch_types=[
            pltpu.VMEM((NBUF, K, SUB), jnp.int32),
            pltpu.VMEM((NBUF, CHUNK, D_MODEL), jnp.float32),
            pltpu.SemaphoreType.DMA,
            pltpu.SemaphoreType.DMA,
            pltpu.SemaphoreType.DMA,
        ],
        compiler_params=pltpu.CompilerParams(use_tc_tiling_on_sc=False),
    )
    def k(table_hbm, idx_hbm, out_hbm, idx_v, rows_v, gsem, osem0, osem1):
        osems = (osem0, osem1)
        wid = lax.axis_index("s") * 2 + lax.axis_index("c")
        base = wid * (per_w // SUB)

        def outer(t, carry):
            for b in range(NBUF):
                row0 = base + (t * NBUF + b) * K

                @pl.when(t > 0)
                def _wait_prev_scatter():
                    pltpu.make_async_copy(
                        rows_v.at[b], out_hbm.at[pl.ds(0, CHUNK)], osems[b]
                    ).wait()

                pltpu.sync_copy(idx_hbm.at[pl.ds(row0, K)], idx_v.at[b])
                descs = [
                    pltpu.async_copy(
                        table_hbm.at[idx_v.at[b].at[j]],
                        rows_v.at[b].at[pl.ds(j * SUB, SUB)],
                        gsem,
                    )
                    for j in range(K)
                ]
                for d in descs:
                    d.wait()
                pltpu.async_copy(
                    rows_v.at[b], out_hbm.at[pl.ds(row0 * SUB, CHUNK)], osems[b]
                )
            return carry

        lax.fori_loop(0, n_ch // NBUF, outer, 0)
        for b in range(NBUF):
            pltpu.make_async_copy(
                rows_v.at[b], out_hbm.at[pl.ds(0, CHUNK)], osems[b]
            ).wait()

    return k


def kernel(cumulative_positions, position_embeddings):
    b, h = cumulative_positions.shape
    n = b * h
    idx2d = cumulative_positions.astype(jnp.int32).reshape(n // SUB, SUB)
    out = _make_gather(n)(position_embeddings, idx2d)
    return out.reshape(b, h, D_MODEL)


# speedup vs baseline: 3.4000x; 1.2124x over previous
import functools

import jax
import jax.numpy as jnp
from jax import lax
from jax.experimental import pallas as pl
from jax.experimental.pallas import tpu as pltpu
from jax.experimental.pallas import tpu_sc as plsc

WIDTH = 128
NUM_WORKERS = 32
SUB = 128
K = 2
CHUNK = SUB * K
NBUF = 2


def _make_gather(m_rows):
    per_w = m_rows // NUM_WORKERS
    n_ch = per_w // CHUNK
    assert per_w % (CHUNK * NBUF) == 0
    mesh = plsc.VectorSubcoreMesh(core_axis_name="c", subcore_axis_name="s")

    @functools.partial(
        pl.kernel,
        out_type=jax.ShapeDtypeStruct((m_rows, WIDTH), jnp.float32),
        mesh=mesh,
        scratch_types=[
            pltpu.VMEM((NBUF, K, SUB), jnp.int32),
            pltpu.VMEM((NBUF, CHUNK, WIDTH), jnp.float32),
            pltpu.SemaphoreType.DMA,
            pltpu.SemaphoreType.DMA,
            pltpu.SemaphoreType.DMA,
        ],
        compiler_params=pltpu.CompilerParams(use_tc_tiling_on_sc=True),
    )
    def k(table_hbm, idx_hbm, out_hbm, idx_v, rows_v, gsem, osem0, osem1):
        osems = (osem0, osem1)
        wid = lax.axis_index("s") * 2 + lax.axis_index("c")
        base = wid * (per_w // SUB)

        def outer(t, carry):
            for b in range(NBUF):
                row0 = base + (t * NBUF + b) * K

                @pl.when(t > 0)
                def _wait_prev_scatter():
                    pltpu.make_async_copy(
                        rows_v.at[b], out_hbm.at[pl.ds(0, CHUNK)], osems[b]
                    ).wait()

                pltpu.sync_copy(idx_hbm.at[pl.ds(row0, K)], idx_v.at[b])
                descs = [
                    pltpu.async_copy(
                        table_hbm.at[idx_v.at[b].at[j]],
                        rows_v.at[b].at[pl.ds(j * SUB, SUB)],
                        gsem,
                    )
                    for j in range(K)
                ]
                for d in descs:
                    d.wait()
                pltpu.async_copy(
                    rows_v.at[b], out_hbm.at[pl.ds(row0 * SUB, CHUNK)], osems[b]
                )
            return carry

        lax.fori_loop(0, n_ch // NBUF, outer, 0)
        for b in range(NBUF):
            pltpu.make_async_copy(
                rows_v.at[b], out_hbm.at[pl.ds(0, CHUNK)], osems[b]
            ).wait()

    return k


def kernel(cumulative_positions, position_embeddings):
    b, h = cumulative_positions.shape
    n = b * h
    v = position_embeddings.shape[0]
    d = position_embeddings.shape[1]
    flat = cumulative_positions.astype(jnp.int32).reshape(n)
    pair_idx = flat[0::2] * v + flat[1::2]
    left = jnp.broadcast_to(position_embeddings[:, None, :], (v, v, d))
    right = jnp.broadcast_to(position_embeddings[None, :, :], (v, v, d))
    t2 = jnp.concatenate([left, right], axis=-1).reshape(v * v, 2 * d)
    idx2d = pair_idx.reshape(n // 2 // SUB, SUB)
    out = _make_gather(n // 2)(t2, idx2d)
    return out.reshape(b, h, d)
